# Initial kernel scaffold; baseline (speedup 1.0000x reference)
import jax
import jax.numpy as jnp
from jax.experimental import pallas as pl


def kernel(pred_verts, gt_verts, flag, faces):
    raise NotImplementedError("write your pallas kernel here")


# speedup vs baseline: 3.1066x; 3.1066x over previous
import functools

import jax
import jax.numpy as jnp
from jax import lax
from jax.experimental import pallas as pl
from jax.experimental.pallas import tpu as pltpu
from jax.experimental.pallas import tpu_sc as plsc

N_VERTS = 6890
N_FACES = 13776
B = 64

NC = 2
NS = 16
NW = NC * NS
L = 16

K = 40
ITERS = 11
ROWD = 6 * B
NB = B // L


def _edge_body(table_hbm, idxs_hbm, mask_hbm, out_hbm,
               idx_v, buf_v, mask_v, acc_v, out_v, sem0, sem1):
    cid = lax.axis_index("c")
    sid = lax.axis_index("s")
    w = sid * NC + cid

    pltpu.sync_copy(idxs_hbm.at[w], idx_v)
    pltpu.sync_copy(mask_hbm, mask_v)
    for cc in range(NB):
        acc_v[cc, :] = jnp.zeros((L,), jnp.float32)

    sems = (sem0, sem1)
    pending = pltpu.async_copy(table_hbm.at[idx_v.at[0]], buf_v.at[0], sem0)
    for it in range(ITERS):
        slot = it % 2
        cur = pending
        if it + 1 < ITERS:
            pending = pltpu.async_copy(
                table_hbm.at[idx_v.at[it + 1]],
                buf_v.at[(it + 1) % 2],
                sems[(it + 1) % 2],
            )
        cur.wait()

        def face_body(k, _, slot=slot):
            for cc in range(NB):
                o = cc * L
                v1 = [buf_v[slot, k, pl.ds(d * B + o, L)] for d in range(6)]
                v2 = [buf_v[slot, K + k, pl.ds(d * B + o, L)] for d in range(6)]
                v3 = [buf_v[slot, 2 * K + k, pl.ds(d * B + o, L)] for d in range(6)]
                e12p = (jnp.abs(v1[0] - v2[0]) + jnp.abs(v1[1] - v2[1])
                        + jnp.abs(v1[2] - v2[2]))
                e13p = (jnp.abs(v1[0] - v3[0]) + jnp.abs(v1[1] - v3[1])
                        + jnp.abs(v1[2] - v3[2]))
                e23p = (jnp.abs(v2[0] - v3[0]) + jnp.abs(v2[1] - v3[1])
                        + jnp.abs(v2[2] - v3[2]))
                e12g = (jnp.abs(v1[3] - v2[3]) + jnp.abs(v1[4] - v2[4])
                        + jnp.abs(v1[5] - v2[5]))
                e13g = (jnp.abs(v1[3] - v3[3]) + jnp.abs(v1[4] - v3[4])
                        + jnp.abs(v1[5] - v3[5]))
                e23g = (jnp.abs(v2[3] - v3[3]) + jnp.abs(v2[4] - v3[4])
                        + jnp.abs(v2[5] - v3[5]))
                t = (jnp.abs(e12p - e12g) + jnp.abs(e13p - e13g)
                     + jnp.abs(e23p - e23g))
                plsc.addupdate(acc_v.at[cc], t)
            return 0

        lax.fori_loop(0, K, face_body, 0)

    part = acc_v[0, :] * mask_v[pl.ds(0, L)]
    msum = mask_v[pl.ds(0, L)]
    for cc in range(1, NB):
        part = part + acc_v[cc, :] * mask_v[pl.ds(cc * L, L)]
        msum = msum + mask_v[pl.ds(cc * L, L)]
    cs = jnp.flip(plsc.cumsum(msum))
    lane = lax.iota(jnp.int32, L)
    total = plsc.cumsum(jnp.where(lane == 0, cs, jnp.float32(0.0)))
    denom = total * jnp.float32(N_FACES)
    out_v[...] = part / denom
    pltpu.sync_copy(out_v, out_hbm.at[w])


@jax.jit
def _edge_loss(table, idxs, maskf):
    mesh = plsc.VectorSubcoreMesh(core_axis_name="c", subcore_axis_name="s")
    run = functools.partial(
        pl.kernel,
        out_type=jax.ShapeDtypeStruct((NW, L), jnp.float32),
        mesh=mesh,
        compiler_params=pltpu.CompilerParams(needs_layout_passes=False),
        scratch_types=[
            pltpu.VMEM((ITERS, 3 * K), jnp.int32),
            pltpu.VMEM((2, 3 * K, ROWD), jnp.float32),
            pltpu.VMEM((B,), jnp.float32),
            pltpu.VMEM((NB, L), jnp.float32),
            pltpu.VMEM((L,), jnp.float32),
            pltpu.SemaphoreType.DMA,
            pltpu.SemaphoreType.DMA,
        ],
    )(_edge_body)
    out = run(table, idxs, maskf)
    return jnp.sum(out)


def kernel(pred_verts, gt_verts, flag, faces):
    table = jnp.concatenate(
        [pred_verts.transpose(1, 2, 0).reshape(N_VERTS, 3 * B),
         gt_verts.transpose(1, 2, 0).reshape(N_VERTS, 3 * B)], axis=1)
    f = faces.astype(jnp.int32)
    pad = NW * ITERS * K - N_FACES
    fp = jnp.concatenate([f, jnp.zeros((pad, 3), jnp.int32)], axis=0)
    idxs = (fp.reshape(NW, ITERS, K, 3)
            .transpose(0, 1, 3, 2)
            .reshape(NW, ITERS, 3 * K))
    maskf = (flag == 1).astype(jnp.float32)
    return _edge_loss(table, idxs, maskf)
